# baseline (speedup 1.0000x reference)
import jax
import jax.numpy as jnp
from jax.experimental import pallas as pl
from jax.experimental.pallas import tpu as pltpu

EPS = 1e-5
BN = (1.0 + EPS) ** -0.5
NND = 10000
EMBD = 128
HIDD = 256


def _enc_body(x_ref, w1_ref, b1_ref, g1_ref, be1_ref, w2_ref, b2_ref, g2_ref,
              be2_ref, z_ref):
    h = jnp.dot(x_ref[...], w1_ref[...], preferred_element_type=jnp.float32)
    h = g1_ref[...] * ((h + b1_ref[...]) * BN) + be1_ref[...]
    h = jnp.maximum(h, 0.0)
    z = jnp.dot(h, w2_ref[...], preferred_element_type=jnp.float32)
    z_ref[...] = g2_ref[...] * ((z + b2_ref[...]) * BN) + be2_ref[...]


def _encode(x, W1, b1, g1, be1, W2, b2, g2, be2):
    n, d = x.shape
    rb = 1000
    full = lambda r: (0, 0)
    return pl.pallas_call(
        _enc_body,
        grid=(n // rb,),
        in_specs=[
            pl.BlockSpec((rb, d), lambda r: (r, 0)),
            pl.BlockSpec((d, HIDD), full),
            pl.BlockSpec((1, HIDD), full),
            pl.BlockSpec((1, HIDD), full),
            pl.BlockSpec((1, HIDD), full),
            pl.BlockSpec((HIDD, EMBD), full),
            pl.BlockSpec((1, EMBD), full),
            pl.BlockSpec((1, EMBD), full),
            pl.BlockSpec((1, EMBD), full),
        ],
        out_specs=pl.BlockSpec((rb, EMBD), lambda r: (r, 0)),
        out_shape=jax.ShapeDtypeStruct((n, EMBD), jnp.float32),
    )(x, W1, b1.reshape(1, -1), g1.reshape(1, -1), be1.reshape(1, -1),
      W2, b2.reshape(1, -1), g2.reshape(1, -1), be2.reshape(1, -1))


def _deg_body(adj_ref, dinv_ref):
    i = pl.program_id(0)
    ni = pl.num_programs(0)
    s = jnp.sum(adj_ref[...], axis=0, keepdims=True)

    @pl.when(i == 0)
    def _():
        dinv_ref[...] = s

    @pl.when(i > 0)
    def _():
        dinv_ref[...] += s

    @pl.when(i == ni - 1)
    def _():
        dinv_ref[...] = jax.lax.rsqrt(dinv_ref[...] + 1.0)


def _degree(adj):
    n = adj.shape[0]
    rb = 200
    return pl.pallas_call(
        _deg_body,
        grid=(n // rb,),
        in_specs=[pl.BlockSpec((rb, n), lambda i: (i, 0))],
        out_specs=pl.BlockSpec((1, n), lambda i: (0, 0)),
        out_shape=jax.ShapeDtypeStruct((1, n), jnp.float32),
    )(adj)


def _hprep_body(x_ref, w_ref, dinv_ref, h_ref, hs_ref):
    h = jnp.dot(x_ref[...], w_ref[...], preferred_element_type=jnp.float32)
    h_ref[...] = h
    hs_ref[...] = h * dinv_ref[...]


def _hprep(x, W, dinv_col):
    n = x.shape[0]
    rb = 1000
    return pl.pallas_call(
        _hprep_body,
        grid=(n // rb,),
        in_specs=[
            pl.BlockSpec((rb, EMBD), lambda r: (r, 0)),
            pl.BlockSpec((EMBD, EMBD), lambda r: (0, 0)),
            pl.BlockSpec((rb, 1), lambda r: (r, 0)),
        ],
        out_specs=[
            pl.BlockSpec((rb, EMBD), lambda r: (r, 0)),
            pl.BlockSpec((rb, EMBD), lambda r: (r, 0)),
        ],
        out_shape=[
            jax.ShapeDtypeStruct((n, EMBD), jnp.float32),
            jax.ShapeDtypeStruct((n, EMBD), jnp.float32),
        ],
    )(x, W, dinv_col)


def _gcn_mm_body(adj_ref, hs_ref, h_ref, dinv_ref, b_ref, out_ref, acc_ref):
    i = pl.program_id(0)
    ni = pl.num_programs(0)
    part = jax.lax.dot_general(
        hs_ref[...], adj_ref[...], (((0,), (0,)), ((), ())),
        preferred_element_type=jnp.float32)

    @pl.when(i == 0)
    def _():
        acc_ref[...] = part

    @pl.when(i > 0)
    def _():
        acc_ref[...] += part

    @pl.when(i == ni - 1)
    def _():
        dv = dinv_ref[...]
        agg = acc_ref[...].T
        out_ref[...] = jnp.maximum(
            agg * dv + h_ref[...] * (dv * dv) + b_ref[...], 0.0)


def _gcn_mm(adj, hs, h, dinv_col, b_row):
    n = adj.shape[0]
    rb = 200
    return pl.pallas_call(
        _gcn_mm_body,
        grid=(n // rb,),
        in_specs=[
            pl.BlockSpec((rb, n), lambda i: (i, 0)),
            pl.BlockSpec((rb, EMBD), lambda i: (i, 0)),
            pl.BlockSpec((n, EMBD), lambda i: (0, 0)),
            pl.BlockSpec((n, 1), lambda i: (0, 0)),
            pl.BlockSpec((1, EMBD), lambda i: (0, 0)),
        ],
        out_specs=pl.BlockSpec((n, EMBD), lambda i: (0, 0)),
        out_shape=jax.ShapeDtypeStruct((n, EMBD), jnp.float32),
        scratch_shapes=[pltpu.VMEM((EMBD, n), jnp.float32)],
    )(adj, hs, h, dinv_col, b_row)


def _head_body(x_ref, w1_ref, b1_ref, w2_ref, b2_ref, o_ref, acc_ref):
    r = pl.program_id(0)
    nr = pl.num_programs(0)
    s = jnp.sum(x_ref[...], axis=0, keepdims=True)

    @pl.when(r == 0)
    def _():
        acc_ref[...] = s

    @pl.when(r > 0)
    def _():
        acc_ref[...] += s

    @pl.when(r == nr - 1)
    def _():
        g = acc_ref[...] * (1.0 / NND)
        hh = jnp.dot(g, w1_ref[...], preferred_element_type=jnp.float32)
        hh = jnp.maximum(hh + b1_ref[...], 0.0)
        o_ref[...] = jnp.dot(
            hh, w2_ref[...], preferred_element_type=jnp.float32) + b2_ref[...]


def _head(x, W1, b1, W2, b2):
    n = x.shape[0]
    rb = 1000
    full = lambda r: (0, 0)
    return pl.pallas_call(
        _head_body,
        grid=(n // rb,),
        in_specs=[
            pl.BlockSpec((rb, EMBD), lambda r: (r, 0)),
            pl.BlockSpec((EMBD, HIDD), full),
            pl.BlockSpec((1, HIDD), full),
            pl.BlockSpec((HIDD, 1), full),
            pl.BlockSpec((1, 1), full),
        ],
        out_specs=pl.BlockSpec((1, 1), full),
        out_shape=jax.ShapeDtypeStruct((1, 1), jnp.float32),
        scratch_shapes=[pltpu.VMEM((1, EMBD), jnp.float32)],
    )(x, W1, b1.reshape(1, -1), W2, b2.reshape(1, -1))


def kernel(rna, methylation, adjacency_matrix, rna_W1, rna_b1, rna_g1, rna_be1,
           rna_W2, rna_b2, rna_g2, rna_be2, meth_W1, meth_b1, meth_g1,
           meth_be1, meth_W2, meth_b2, meth_g2, meth_be2, gcn1_W, gcn1_b,
           gcn2_W, gcn2_b, pred_W1, pred_b1, pred_W2, pred_b2):
    z1 = _encode(rna, rna_W1, rna_b1, rna_g1, rna_be1, rna_W2, rna_b2, rna_g2,
                 rna_be2)
    z2 = _encode(methylation, meth_W1, meth_b1, meth_g1, meth_be1, meth_W2,
                 meth_b2, meth_g2, meth_be2)
    x = jnp.concatenate([z1, z2], axis=0)
    dinv_col = _degree(adjacency_matrix).reshape(-1, 1)
    for W, b in ((gcn1_W, gcn1_b), (gcn2_W, gcn2_b)):
        h, hs = _hprep(x, W, dinv_col)
        x = _gcn_mm(adjacency_matrix, hs, h, dinv_col, b.reshape(1, -1))
    out = _head(x, pred_W1, pred_b1, pred_W2, pred_b2)
    return out.reshape(1)


# speedup vs baseline: 1.0049x; 1.0049x over previous
import jax
import jax.numpy as jnp
from jax.experimental import pallas as pl
from jax.experimental.pallas import tpu as pltpu

EPS = 1e-5
BN = (1.0 + EPS) ** -0.5
NND = 10000
EMBD = 128
HIDD = 256


def _enc_body(x_ref, w1_ref, b1_ref, g1_ref, be1_ref, w2_ref, b2_ref, g2_ref,
              be2_ref, z_ref):
    h = jnp.dot(x_ref[...], w1_ref[...], preferred_element_type=jnp.float32)
    h = g1_ref[...] * ((h + b1_ref[...]) * BN) + be1_ref[...]
    h = jnp.maximum(h, 0.0)
    z = jnp.dot(h, w2_ref[...], preferred_element_type=jnp.float32)
    z_ref[...] = g2_ref[...] * ((z + b2_ref[...]) * BN) + be2_ref[...]


def _encode(x, W1, b1, g1, be1, W2, b2, g2, be2):
    n, d = x.shape
    rb = 1000
    full = lambda r: (0, 0)
    return pl.pallas_call(
        _enc_body,
        grid=(n // rb,),
        in_specs=[
            pl.BlockSpec((rb, d), lambda r: (r, 0)),
            pl.BlockSpec((d, HIDD), full),
            pl.BlockSpec((1, HIDD), full),
            pl.BlockSpec((1, HIDD), full),
            pl.BlockSpec((1, HIDD), full),
            pl.BlockSpec((HIDD, EMBD), full),
            pl.BlockSpec((1, EMBD), full),
            pl.BlockSpec((1, EMBD), full),
            pl.BlockSpec((1, EMBD), full),
        ],
        out_specs=pl.BlockSpec((rb, EMBD), lambda r: (r, 0)),
        out_shape=jax.ShapeDtypeStruct((n, EMBD), jnp.float32),
    )(x, W1, b1.reshape(1, -1), g1.reshape(1, -1), be1.reshape(1, -1),
      W2, b2.reshape(1, -1), g2.reshape(1, -1), be2.reshape(1, -1))


def _pack_body(adj_ref, abf_ref, dinv_ref):
    i = pl.program_id(0)
    ni = pl.num_programs(0)
    a = adj_ref[...]
    abf_ref[...] = a.astype(jnp.bfloat16)
    s = jnp.sum(a, axis=0, keepdims=True)

    @pl.when(i == 0)
    def _():
        dinv_ref[...] = s

    @pl.when(i > 0)
    def _():
        dinv_ref[...] += s

    @pl.when(i == ni - 1)
    def _():
        dinv_ref[...] = jax.lax.rsqrt(dinv_ref[...] + 1.0)


def _pack(adj):
    n = adj.shape[0]
    rb = 200
    return pl.pallas_call(
        _pack_body,
        grid=(n // rb,),
        in_specs=[pl.BlockSpec((rb, n), lambda i: (i, 0))],
        out_specs=[
            pl.BlockSpec((rb, n), lambda i: (i, 0)),
            pl.BlockSpec((1, n), lambda i: (0, 0)),
        ],
        out_shape=[
            jax.ShapeDtypeStruct((n, n), jnp.bfloat16),
            jax.ShapeDtypeStruct((1, n), jnp.float32),
        ],
    )(adj)


def _hprep_body(x_ref, w_ref, dinv_ref, h_ref, hs_ref):
    h = jnp.dot(x_ref[...], w_ref[...], preferred_element_type=jnp.float32)
    h_ref[...] = h
    hs_ref[...] = (h * dinv_ref[...]).astype(jnp.bfloat16)


def _hprep(x, W, dinv_col):
    n = x.shape[0]
    rb = 1000
    return pl.pallas_call(
        _hprep_body,
        grid=(n // rb,),
        in_specs=[
            pl.BlockSpec((rb, EMBD), lambda r: (r, 0)),
            pl.BlockSpec((EMBD, EMBD), lambda r: (0, 0)),
            pl.BlockSpec((rb, 1), lambda r: (r, 0)),
        ],
        out_specs=[
            pl.BlockSpec((rb, EMBD), lambda r: (r, 0)),
            pl.BlockSpec((rb, EMBD), lambda r: (r, 0)),
        ],
        out_shape=[
            jax.ShapeDtypeStruct((n, EMBD), jnp.float32),
            jax.ShapeDtypeStruct((n, EMBD), jnp.bfloat16),
        ],
    )(x, W, dinv_col)


def _gcn_mm_body(adj_ref, hs_ref, h_ref, dinv_ref, b_ref, out_ref, acc_ref):
    i = pl.program_id(0)
    ni = pl.num_programs(0)
    part = jax.lax.dot_general(
        hs_ref[...], adj_ref[...], (((0,), (0,)), ((), ())),
        preferred_element_type=jnp.float32)

    @pl.when(i == 0)
    def _():
        acc_ref[...] = part

    @pl.when(i > 0)
    def _():
        acc_ref[...] += part

    @pl.when(i == ni - 1)
    def _():
        dv = dinv_ref[...]
        agg = acc_ref[...].T
        out_ref[...] = jnp.maximum(
            agg * dv + h_ref[...] * (dv * dv) + b_ref[...], 0.0)


def _gcn_mm(adj, hs, h, dinv_col, b_row):
    n = adj.shape[0]
    rb = 200
    return pl.pallas_call(
        _gcn_mm_body,
        grid=(n // rb,),
        in_specs=[
            pl.BlockSpec((rb, n), lambda i: (i, 0)),
            pl.BlockSpec((rb, EMBD), lambda i: (i, 0)),
            pl.BlockSpec((n, EMBD), lambda i: (0, 0)),
            pl.BlockSpec((n, 1), lambda i: (0, 0)),
            pl.BlockSpec((1, EMBD), lambda i: (0, 0)),
        ],
        out_specs=pl.BlockSpec((n, EMBD), lambda i: (0, 0)),
        out_shape=jax.ShapeDtypeStruct((n, EMBD), jnp.float32),
        scratch_shapes=[pltpu.VMEM((EMBD, n), jnp.float32)],
    )(adj, hs, h, dinv_col, b_row)


def _head_body(x_ref, w1_ref, b1_ref, w2_ref, b2_ref, o_ref, acc_ref):
    r = pl.program_id(0)
    nr = pl.num_programs(0)
    s = jnp.sum(x_ref[...], axis=0, keepdims=True)

    @pl.when(r == 0)
    def _():
        acc_ref[...] = s

    @pl.when(r > 0)
    def _():
        acc_ref[...] += s

    @pl.when(r == nr - 1)
    def _():
        g = acc_ref[...] * (1.0 / NND)
        hh = jnp.dot(g, w1_ref[...], preferred_element_type=jnp.float32)
        hh = jnp.maximum(hh + b1_ref[...], 0.0)
        o_ref[...] = jnp.dot(
            hh, w2_ref[...], preferred_element_type=jnp.float32) + b2_ref[...]


def _head(x, W1, b1, W2, b2):
    n = x.shape[0]
    rb = 1000
    full = lambda r: (0, 0)
    return pl.pallas_call(
        _head_body,
        grid=(n // rb,),
        in_specs=[
            pl.BlockSpec((rb, EMBD), lambda r: (r, 0)),
            pl.BlockSpec((EMBD, HIDD), full),
            pl.BlockSpec((1, HIDD), full),
            pl.BlockSpec((HIDD, 1), full),
            pl.BlockSpec((1, 1), full),
        ],
        out_specs=pl.BlockSpec((1, 1), full),
        out_shape=jax.ShapeDtypeStruct((1, 1), jnp.float32),
        scratch_shapes=[pltpu.VMEM((1, EMBD), jnp.float32)],
    )(x, W1, b1.reshape(1, -1), W2, b2.reshape(1, -1))


def kernel(rna, methylation, adjacency_matrix, rna_W1, rna_b1, rna_g1, rna_be1,
           rna_W2, rna_b2, rna_g2, rna_be2, meth_W1, meth_b1, meth_g1,
           meth_be1, meth_W2, meth_b2, meth_g2, meth_be2, gcn1_W, gcn1_b,
           gcn2_W, gcn2_b, pred_W1, pred_b1, pred_W2, pred_b2):
    z1 = _encode(rna, rna_W1, rna_b1, rna_g1, rna_be1, rna_W2, rna_b2, rna_g2,
                 rna_be2)
    z2 = _encode(methylation, meth_W1, meth_b1, meth_g1, meth_be1, meth_W2,
                 meth_b2, meth_g2, meth_be2)
    x = jnp.concatenate([z1, z2], axis=0)
    abf, dinv = _pack(adjacency_matrix)
    dinv_col = dinv.reshape(-1, 1)
    for W, b in ((gcn1_W, gcn1_b), (gcn2_W, gcn2_b)):
        h, hs = _hprep(x, W, dinv_col)
        x = _gcn_mm(abf, hs, h, dinv_col, b.reshape(1, -1))
    out = _head(x, pred_W1, pred_b1, pred_W2, pred_b2)
    return out.reshape(1)


# speedup vs baseline: 1.0727x; 1.0675x over previous
import jax
import jax.numpy as jnp
from jax.experimental import pallas as pl
from jax.experimental.pallas import tpu as pltpu

EPS = 1e-5
BN = (1.0 + EPS) ** -0.5
NND = 10000
EMBD = 128
HIDD = 256


def _enc_body(x_ref, w1_ref, b1_ref, g1_ref, be1_ref, w2_ref, b2_ref, g2_ref,
              be2_ref, z_ref):
    h = jnp.dot(x_ref[...], w1_ref[...], preferred_element_type=jnp.float32)
    h = g1_ref[...] * ((h + b1_ref[...]) * BN) + be1_ref[...]
    h = jnp.maximum(h, 0.0)
    z = jnp.dot(h, w2_ref[...], preferred_element_type=jnp.float32)
    z_ref[...] = g2_ref[...] * ((z + b2_ref[...]) * BN) + be2_ref[...]


def _encode(x, W1, b1, g1, be1, W2, b2, g2, be2):
    n, d = x.shape
    rb = 1000
    full = lambda r: (0, 0)
    return pl.pallas_call(
        _enc_body,
        grid=(n // rb,),
        in_specs=[
            pl.BlockSpec((rb, d), lambda r: (r, 0)),
            pl.BlockSpec((d, HIDD), full),
            pl.BlockSpec((1, HIDD), full),
            pl.BlockSpec((1, HIDD), full),
            pl.BlockSpec((1, HIDD), full),
            pl.BlockSpec((HIDD, EMBD), full),
            pl.BlockSpec((1, EMBD), full),
            pl.BlockSpec((1, EMBD), full),
            pl.BlockSpec((1, EMBD), full),
        ],
        out_specs=pl.BlockSpec((rb, EMBD), lambda r: (r, 0)),
        out_shape=jax.ShapeDtypeStruct((n, EMBD), jnp.float32),
    )(x, W1, b1.reshape(1, -1), g1.reshape(1, -1), be1.reshape(1, -1),
      W2, b2.reshape(1, -1), g2.reshape(1, -1), be2.reshape(1, -1))


def _pack_body(adj_ref, abf_ref, dinv_ref):
    i = pl.program_id(0)
    ni = pl.num_programs(0)
    a = adj_ref[...]
    abf_ref[...] = a.astype(jnp.bfloat16)
    s = jnp.sum(a, axis=0, keepdims=True)

    @pl.when(i == 0)
    def _():
        dinv_ref[...] = s

    @pl.when(i > 0)
    def _():
        dinv_ref[...] += s

    @pl.when(i == ni - 1)
    def _():
        dinv_ref[...] = jax.lax.rsqrt(dinv_ref[...] + 1.0)


def _pack(adj):
    n = adj.shape[0]
    rb = 200
    return pl.pallas_call(
        _pack_body,
        grid=(n // rb,),
        in_specs=[pl.BlockSpec((rb, n), lambda i: (i, 0))],
        out_specs=[
            pl.BlockSpec((rb, n), lambda i: (i, 0)),
            pl.BlockSpec((1, n), lambda i: (0, 0)),
        ],
        out_shape=[
            jax.ShapeDtypeStruct((n, n), jnp.bfloat16),
            jax.ShapeDtypeStruct((1, n), jnp.float32),
        ],
    )(adj)


def _gcn_mm_body(adj_ref, x_ref, w_ref, dinv_ref, b_ref, pw1_ref, pb1_ref,
                 pw2_ref, pb2_ref, out_ref, ho_ref, acc_ref, h_ref, hs_ref):
    i = pl.program_id(0)
    ni = pl.num_programs(0)
    rb = adj_ref.shape[0]

    @pl.when(i == 0)
    def _():
        h = jnp.dot(x_ref[...], w_ref[...], preferred_element_type=jnp.float32)
        h_ref[...] = h
        hs_ref[...] = (h * dinv_ref[...]).astype(jnp.bfloat16)

    part = jax.lax.dot_general(
        hs_ref[pl.ds(i * rb, rb), :], adj_ref[...], (((0,), (0,)), ((), ())),
        preferred_element_type=jnp.float32)

    @pl.when(i == 0)
    def _():
        acc_ref[...] = part

    @pl.when(i > 0)
    def _():
        acc_ref[...] += part

    @pl.when(i == ni - 1)
    def _():
        dv = dinv_ref[...]
        agg = acc_ref[...].T
        xo = jnp.maximum(
            agg * dv + h_ref[...] * (dv * dv) + b_ref[...], 0.0)
        out_ref[...] = xo
        g = jnp.sum(xo, axis=0, keepdims=True) * (1.0 / NND)
        hh = jnp.dot(g, pw1_ref[...], preferred_element_type=jnp.float32)
        hh = jnp.maximum(hh + pb1_ref[...], 0.0)
        ho_ref[...] = jnp.sum(hh * pw2_ref[...], axis=1, keepdims=True) \
            + pb2_ref[...]


def _gcn_mm(adj, x, W, dinv_col, b_row, pW1, pb1, pW2_row, pb2):
    n = x.shape[0]
    rb = 200
    full = lambda i: (0, 0)
    return pl.pallas_call(
        _gcn_mm_body,
        grid=(n // rb,),
        in_specs=[
            pl.BlockSpec((rb, n), lambda i: (i, 0)),
            pl.BlockSpec((n, EMBD), full),
            pl.BlockSpec((EMBD, EMBD), full),
            pl.BlockSpec((n, 1), full),
            pl.BlockSpec((1, EMBD), full),
            pl.BlockSpec((EMBD, HIDD), full),
            pl.BlockSpec((1, HIDD), full),
            pl.BlockSpec((1, HIDD), full),
            pl.BlockSpec((1, 1), full),
        ],
        out_specs=[
            pl.BlockSpec((n, EMBD), full),
            pl.BlockSpec((1, 1), full),
        ],
        out_shape=[
            jax.ShapeDtypeStruct((n, EMBD), jnp.float32),
            jax.ShapeDtypeStruct((1, 1), jnp.float32),
        ],
        scratch_shapes=[
            pltpu.VMEM((EMBD, n), jnp.float32),
            pltpu.VMEM((n, EMBD), jnp.float32),
            pltpu.VMEM((n, EMBD), jnp.bfloat16),
        ],
    )(adj, x, W, dinv_col, b_row, pW1, pb1, pW2_row, pb2)


def kernel(rna, methylation, adjacency_matrix, rna_W1, rna_b1, rna_g1, rna_be1,
           rna_W2, rna_b2, rna_g2, rna_be2, meth_W1, meth_b1, meth_g1,
           meth_be1, meth_W2, meth_b2, meth_g2, meth_be2, gcn1_W, gcn1_b,
           gcn2_W, gcn2_b, pred_W1, pred_b1, pred_W2, pred_b2):
    z1 = _encode(rna, rna_W1, rna_b1, rna_g1, rna_be1, rna_W2, rna_b2, rna_g2,
                 rna_be2)
    z2 = _encode(methylation, meth_W1, meth_b1, meth_g1, meth_be1, meth_W2,
                 meth_b2, meth_g2, meth_be2)
    x = jnp.concatenate([z1, z2], axis=0)
    abf, dinv = _pack(adjacency_matrix)
    dinv_col = dinv.reshape(-1, 1)
    pw2_row = pred_W2.reshape(1, -1)
    pb2_row = pred_b2.reshape(1, 1)
    pb1_row = pred_b1.reshape(1, -1)
    out = None
    for W, b in ((gcn1_W, gcn1_b), (gcn2_W, gcn2_b)):
        x, out = _gcn_mm(abf, x, W, dinv_col, b.reshape(1, -1),
                         pred_W1, pb1_row, pw2_row, pb2_row)
    return out.reshape(1)
